# Initial kernel scaffold; baseline (speedup 1.0000x reference)
import jax
import jax.numpy as jnp
from jax.experimental import pallas as pl


def kernel(x, edge_index, edge_type, basis1, comp1, root1, bias1, basis2, comp2, root2, bias2):
    raise NotImplementedError("write your pallas kernel here")


# speedup vs baseline: 1.7234x; 1.7234x over previous
import functools

import jax
import jax.numpy as jnp
from jax import lax
from jax.experimental import pallas as pl
from jax.experimental.pallas import tpu as pltpu
from jax.experimental.pallas import tpu_sc as plsc

N = 10000
E = 160000
R = 8
D = 256

NC = 2
NS = 16
L = 16
HW = 128

NH = N // NC
KH = NH * R
KHP = 40960
KR = KHP // HW
KTRASH = 316
NRT = 32

NROW = 10240
NPT = NROW // NS

EPT = E // NS
SCE = 2048
NSC = 5
K = 64
NCHI = SCE // K
EP2 = (NS - 1) * EPT + NSC * SCE + 64

_mesh = plsc.VectorSubcoreMesh(
    core_axis_name="c", subcore_axis_name="s", num_cores=NC, num_subcores=NS)
_sc_params = pltpu.CompilerParams(needs_layout_passes=False)


@functools.partial(
    pl.kernel,
    out_type=jax.ShapeDtypeStruct((NC * KHP,), jnp.float32),
    mesh=_mesh,
    compiler_params=_sc_params,
    scratch_types=[
        pltpu.VMEM_SHARED((KR, HW), jnp.float32),
        pltpu.VMEM((SCE,), jnp.int32),
        pltpu.VMEM((SCE,), jnp.int32),
        pltpu.VMEM((K,), jnp.int32),
        pltpu.VMEM((K, HW), jnp.float32),
        pltpu.VMEM((NRT, HW), jnp.float32),
        pltpu.VMEM((NRT * HW,), jnp.float32),
    ],
)
def _sc_counts(dst_h, et_h, norm_h, acc_sh, dst_v, et_v, kidx_v, val_v,
               buf_v, nrm_v):
    c = lax.axis_index("c")
    s = lax.axis_index("s")
    lo = c * NH
    zeros16 = jnp.zeros((L,), jnp.float32)
    iota = lax.iota(jnp.int32, L)

    @pl.when(s < KR // NRT)
    def _zero():
        for j in range(NRT):
            for t in range(HW // L):
                buf_v[j, pl.ds(t * L, L)] = zeros16
        pltpu.sync_copy(buf_v, acc_sh.at[pl.ds(s * NRT, NRT)])

    plsc.subcore_barrier()

    def schunk(g, carry):
        ebase = s * EPT + g * SCE
        pltpu.sync_copy(dst_h.at[pl.ds(ebase, SCE)], dst_v)
        pltpu.sync_copy(et_h.at[pl.ds(ebase, SCE)], et_v)

        def chunk(i, carry2):
            base = i * K
            for v in range(K // L):
                off = base + v * L
                d = dst_v[pl.ds(off, L)]
                e = et_v[pl.ds(off, L)]
                slot = g * SCE + off + iota
                loc = d - lo
                valid = (slot < EPT) & (loc >= 0) & (loc < NH)
                lk = loc * R + e
                row = lax.shift_right_logical(lk, 7)
                lane = jnp.where(valid, lk & (HW - 1), 0)
                kidx_v[pl.ds(v * L, L)] = jnp.where(valid, row, KTRASH)
                for j in range(L):
                    bc = jnp.broadcast_to(lane[j], (L,))
                    for t in range(HW // L):
                        val_v[v * L + j, pl.ds(t * L, L)] = jnp.where(
                            iota + t * L == bc, 1.0, 0.0)
            pltpu.sync_copy(val_v, acc_sh.at[kidx_v], add=True)
            return carry2

        lax.fori_loop(0, NCHI, chunk, 0)
        return carry

    lax.fori_loop(0, NSC, schunk, 0)
    plsc.subcore_barrier()

    @pl.when(s < KR // NRT)
    def _readout():
        pltpu.sync_copy(acc_sh.at[pl.ds(s * NRT, NRT)], buf_v)
        for j in range(NRT):
            for t in range(HW // L):
                cnt = buf_v[j, pl.ds(t * L, L)]
                nrm_v[pl.ds(j * HW + t * L, L)] = (
                    1.0 / jnp.maximum(cnt, 1.0))
        pltpu.sync_copy(
            nrm_v, norm_h.at[pl.ds(c * KHP + s * (NRT * HW), NRT * HW)])


EPW = E // (NC * NS)


@functools.partial(
    pl.kernel,
    out_type=jax.ShapeDtypeStruct((EP2,), jnp.float32),
    mesh=_mesh,
    compiler_params=_sc_params,
    scratch_types=[
        pltpu.VMEM((NC * KHP,), jnp.float32),
        pltpu.VMEM((EPW,), jnp.int32),
        pltpu.VMEM((EPW,), jnp.int32),
        pltpu.VMEM((EPW,), jnp.float32),
    ],
)
def _sc_edge_norm(dst_h, et_h, norm_h, en_h, nrm_v, dst_v, et_v, out_v):
    c = lax.axis_index("c")
    s = lax.axis_index("s")
    w = c * NS + s
    pltpu.sync_copy(norm_h, nrm_v)
    pltpu.sync_copy(dst_h.at[pl.ds(w * EPW, EPW)], dst_v)
    pltpu.sync_copy(et_h.at[pl.ds(w * EPW, EPW)], et_v)

    def chunk(i, carry):
        base = jnp.minimum(i * K, EPW - K)
        for v in range(K // L):
            off = base + v * L
            d = dst_v[pl.ds(off, L)]
            e = et_v[pl.ds(off, L)]
            half = jnp.where(d >= NH, 1, 0)
            gk = half * KHP + (d - half * NH) * R + e
            out_v[pl.ds(off, L)] = plsc.load_gather(nrm_v, [gk])
        return carry

    lax.fori_loop(0, (EPW + K - 1) // K, chunk, 0)
    pltpu.sync_copy(out_v, en_h.at[pl.ds(w * EPW, EPW)])


@functools.partial(
    pl.kernel,
    out_type=jax.ShapeDtypeStruct((NC * NROW, HW), jnp.float32),
    mesh=_mesh,
    compiler_params=_sc_params,
    scratch_types=[
        pltpu.VMEM_SHARED((NROW, HW), jnp.float32),
        pltpu.VMEM((SCE,), jnp.int32),
        pltpu.VMEM((SCE,), jnp.int32),
        pltpu.VMEM((SCE,), jnp.int32),
        pltpu.VMEM((SCE,), jnp.float32),
        pltpu.VMEM((K, HW), jnp.float32),
        pltpu.VMEM((K, HW), jnp.float32),
        pltpu.VMEM((K,), jnp.int32),
        pltpu.VMEM((K,), jnp.int32),
        pltpu.VMEM((K,), jnp.int32),
        pltpu.VMEM((K,), jnp.int32),
        pltpu.VMEM((L, HW), jnp.float32),
    ],
)
def _sc_messages(xw2_h, src_h, dst_h, et_h, en_h, aggp_h, acc_sh, src_v,
                 dst_v, et_v, en_v, lo_v, hi_v, gl_v, gh_v, sl_v, sh_v, zb_v):
    c = lax.axis_index("c")
    s = lax.axis_index("s")
    lo = c * NH
    zeros16 = jnp.zeros((L,), jnp.float32)
    iota = lax.iota(jnp.int32, L)
    for j in range(L):
        for t in range(HW // L):
            zb_v[j, pl.ds(t * L, L)] = zeros16

    def zchunk(t, carry):
        pltpu.sync_copy(zb_v, acc_sh.at[pl.ds(s * NPT + t * L, L)])
        return carry

    lax.fori_loop(0, NPT // L, zchunk, 0)
    plsc.subcore_barrier()

    def schunk(g, carry):
        ebase = s * EPT + g * SCE
        pltpu.sync_copy(src_h.at[pl.ds(ebase, SCE)], src_v)
        pltpu.sync_copy(dst_h.at[pl.ds(ebase, SCE)], dst_v)
        pltpu.sync_copy(et_h.at[pl.ds(ebase, SCE)], et_v)
        pltpu.sync_copy(en_h.at[pl.ds(ebase, SCE)], en_v)

        def chunk(i, carry2):
            base = i * K
            for v in range(K // L):
                off = base + v * L
                sr = src_v[pl.ds(off, L)]
                e = et_v[pl.ds(off, L)]
                d = dst_v[pl.ds(off, L)]
                slot = g * SCE + off + iota
                loc = d - lo
                valid = (slot < EPT) & (loc >= 0) & (loc < NH)
                g2 = (e * N + sr) * 2
                gl_v[pl.ds(v * L, L)] = g2
                gh_v[pl.ds(v * L, L)] = g2 + 1
                srow = jnp.where(valid, loc, NH) * 2
                sl_v[pl.ds(v * L, L)] = srow
                sh_v[pl.ds(v * L, L)] = srow + 1
            pltpu.sync_copy(xw2_h.at[gl_v], lo_v)
            pltpu.sync_copy(xw2_h.at[gh_v], hi_v)
            for v in range(K // L):
                nvec = en_v[pl.ds(base + v * L, L)]
                for j in range(L):
                    bc = jnp.broadcast_to(nvec[j], (L,))
                    row = v * L + j
                    for t in range(HW // L):
                        sl = pl.ds(t * L, L)
                        lo_v[row, sl] = lo_v[row, sl] * bc
                        hi_v[row, sl] = hi_v[row, sl] * bc
            pltpu.sync_copy(lo_v, acc_sh.at[sl_v], add=True)
            pltpu.sync_copy(hi_v, acc_sh.at[sh_v], add=True)
            return carry2

        lax.fori_loop(0, NCHI, chunk, 0)
        return carry

    lax.fori_loop(0, NSC, schunk, 0)
    plsc.subcore_barrier()

    def wchunk(t, carry):
        r0 = s * NPT + t * L
        pltpu.sync_copy(acc_sh.at[pl.ds(r0, L)], zb_v)
        pltpu.sync_copy(zb_v, aggp_h.at[pl.ds(c * NROW + r0, L)])
        return carry

    lax.fori_loop(0, NPT // L, wchunk, 0)


def _w_body(comp_ref, basis_ref, w_ref):
    w_ref[...] = jnp.dot(comp_ref[...], basis_ref[...],
                         preferred_element_type=jnp.float32)


def _relation_weights(comp, basis):
    nb = basis.shape[0]
    w2d = pl.pallas_call(
        _w_body,
        out_shape=jax.ShapeDtypeStruct((R, D * D), jnp.float32),
    )(comp, basis.reshape(nb, D * D))
    return w2d.reshape(R, D, D)


BN = 2000


def _xw_body(x_ref, w_ref, o_ref):
    o_ref[0] = jnp.dot(x_ref[...], w_ref[0],
                       preferred_element_type=jnp.float32)


def _all_relation_transform(x, w):
    return pl.pallas_call(
        _xw_body,
        grid=(R, N // BN),
        in_specs=[
            pl.BlockSpec((BN, D), lambda r, j: (j, 0)),
            pl.BlockSpec((1, D, D), lambda r, j: (r, 0, 0)),
        ],
        out_specs=pl.BlockSpec((1, BN, D), lambda r, j: (r, j, 0)),
        out_shape=jax.ShapeDtypeStruct((R, N, D), jnp.float32),
    )(x, w)


def _comb_body(agg_ref, x_ref, root_ref, bias_ref, o_ref):
    acc = agg_ref[...] + jnp.dot(x_ref[...], root_ref[...],
                                 preferred_element_type=jnp.float32)
    o_ref[...] = jnp.maximum(acc + bias_ref[...], 0.0)


def _combine(agg, x, root, bias):
    return pl.pallas_call(
        _comb_body,
        grid=(N // BN,),
        in_specs=[
            pl.BlockSpec((BN, D), lambda j: (j, 0)),
            pl.BlockSpec((BN, D), lambda j: (j, 0)),
            pl.BlockSpec((D, D), lambda j: (0, 0)),
            pl.BlockSpec((1, D), lambda j: (0, 0)),
        ],
        out_specs=pl.BlockSpec((BN, D), lambda j: (j, 0)),
        out_shape=jax.ShapeDtypeStruct((N, D), jnp.float32),
    )(agg, x, root, bias.reshape(1, D))


def _layer(x, srcp, dstp, etp, basis, comp, root, bias, en):
    w = _relation_weights(comp, basis)
    xw = _all_relation_transform(x, w)
    aggp = _sc_messages(xw.reshape(R * N * 2, HW), srcp, dstp, etp, en)
    agg = aggp.reshape(NC, NROW // 2, D)[:, :NH].reshape(N, D)
    return _combine(agg, x, root, bias)


def kernel(x, edge_index, edge_type, basis1, comp1, root1, bias1,
           basis2, comp2, root2, bias2):
    srcp = jnp.pad(edge_index[0], (0, EP2 - E))
    dstp = jnp.pad(edge_index[1], (0, EP2 - E))
    etp = jnp.pad(edge_type, (0, EP2 - E))
    norm = _sc_counts(dstp, etp)
    en = _sc_edge_norm(dstp, etp, norm)
    h = _layer(x, srcp, dstp, etp, basis1, comp1, root1, bias1, en)
    h = _layer(h, srcp, dstp, etp, basis2, comp2, root2, bias2, en)
    return h


# speedup vs baseline: 1.7752x; 1.0301x over previous
import functools

import jax
import jax.numpy as jnp
from jax import lax
from jax.experimental import pallas as pl
from jax.experimental.pallas import tpu as pltpu
from jax.experimental.pallas import tpu_sc as plsc

N = 10000
E = 160000
R = 8
D = 256

NC = 2
NS = 16
L = 16
HW = 128

NH = N // NC
KH = NH * R
KHP = 40960
KR = KHP // HW
KTRASH = 316
NRT = 32

NROW = 10240
NPT = NROW // NS

EPT = E // NS
SCE = 1024
NSC = 10
K = 64
NCHI = SCE // K
EP2 = (NS - 1) * EPT + NSC * SCE + 64

_mesh = plsc.VectorSubcoreMesh(
    core_axis_name="c", subcore_axis_name="s", num_cores=NC, num_subcores=NS)
_sc_params = pltpu.CompilerParams(needs_layout_passes=False)


@functools.partial(
    pl.kernel,
    out_type=jax.ShapeDtypeStruct((NC * KHP,), jnp.float32),
    mesh=_mesh,
    compiler_params=_sc_params,
    scratch_types=[
        pltpu.VMEM_SHARED((KR, HW), jnp.float32),
        pltpu.VMEM((SCE,), jnp.int32),
        pltpu.VMEM((SCE,), jnp.int32),
        pltpu.VMEM((K,), jnp.int32),
        pltpu.VMEM((K, HW), jnp.float32),
        pltpu.VMEM((NRT, HW), jnp.float32),
        pltpu.VMEM((NRT * HW,), jnp.float32),
    ],
)
def _sc_counts(dst_h, et_h, norm_h, acc_sh, dst_v, et_v, kidx_v, val_v,
               buf_v, nrm_v):
    c = lax.axis_index("c")
    s = lax.axis_index("s")
    lo = c * NH
    zeros16 = jnp.zeros((L,), jnp.float32)
    iota = lax.iota(jnp.int32, L)

    @pl.when(s < KR // NRT)
    def _zero():
        for j in range(NRT):
            for t in range(HW // L):
                buf_v[j, pl.ds(t * L, L)] = zeros16
        pltpu.sync_copy(buf_v, acc_sh.at[pl.ds(s * NRT, NRT)])

    plsc.subcore_barrier()

    def schunk(g, carry):
        ebase = s * EPT + g * SCE
        pltpu.sync_copy(dst_h.at[pl.ds(ebase, SCE)], dst_v)
        pltpu.sync_copy(et_h.at[pl.ds(ebase, SCE)], et_v)

        def chunk(i, carry2):
            base = i * K
            for v in range(K // L):
                off = base + v * L
                d = dst_v[pl.ds(off, L)]
                e = et_v[pl.ds(off, L)]
                slot = g * SCE + off + iota
                loc = d - lo
                valid = (slot < EPT) & (loc >= 0) & (loc < NH)
                lk = loc * R + e
                row = lax.shift_right_logical(lk, 7)
                lane = jnp.where(valid, lk & (HW - 1), 0)
                kidx_v[pl.ds(v * L, L)] = jnp.where(valid, row, KTRASH)
                for j in range(L):
                    bc = jnp.broadcast_to(lane[j], (L,))
                    for t in range(HW // L):
                        val_v[v * L + j, pl.ds(t * L, L)] = jnp.where(
                            iota + t * L == bc, 1.0, 0.0)
            pltpu.sync_copy(val_v, acc_sh.at[kidx_v], add=True)
            return carry2

        lax.fori_loop(0, NCHI, chunk, 0)
        return carry

    lax.fori_loop(0, NSC, schunk, 0)
    plsc.subcore_barrier()

    @pl.when(s < KR // NRT)
    def _readout():
        pltpu.sync_copy(acc_sh.at[pl.ds(s * NRT, NRT)], buf_v)
        for j in range(NRT):
            for t in range(HW // L):
                cnt = buf_v[j, pl.ds(t * L, L)]
                nrm_v[pl.ds(j * HW + t * L, L)] = (
                    1.0 / jnp.maximum(cnt, 1.0))
        pltpu.sync_copy(
            nrm_v, norm_h.at[pl.ds(c * KHP + s * (NRT * HW), NRT * HW)])


EPW = E // (NC * NS)


@functools.partial(
    pl.kernel,
    out_type=jax.ShapeDtypeStruct((EP2,), jnp.float32),
    mesh=_mesh,
    compiler_params=_sc_params,
    scratch_types=[
        pltpu.VMEM((NC * KHP,), jnp.float32),
        pltpu.VMEM((EPW,), jnp.int32),
        pltpu.VMEM((EPW,), jnp.int32),
        pltpu.VMEM((EPW,), jnp.float32),
    ],
)
def _sc_edge_norm(dst_h, et_h, norm_h, en_h, nrm_v, dst_v, et_v, out_v):
    c = lax.axis_index("c")
    s = lax.axis_index("s")
    w = c * NS + s
    pltpu.sync_copy(norm_h, nrm_v)
    pltpu.sync_copy(dst_h.at[pl.ds(w * EPW, EPW)], dst_v)
    pltpu.sync_copy(et_h.at[pl.ds(w * EPW, EPW)], et_v)

    def chunk(i, carry):
        base = jnp.minimum(i * K, EPW - K)
        for v in range(K // L):
            off = base + v * L
            d = dst_v[pl.ds(off, L)]
            e = et_v[pl.ds(off, L)]
            half = jnp.where(d >= NH, 1, 0)
            gk = half * KHP + (d - half * NH) * R + e
            out_v[pl.ds(off, L)] = plsc.load_gather(nrm_v, [gk])
        return carry

    lax.fori_loop(0, (EPW + K - 1) // K, chunk, 0)
    pltpu.sync_copy(out_v, en_h.at[pl.ds(w * EPW, EPW)])


@functools.partial(
    pl.kernel,
    out_type=jax.ShapeDtypeStruct((NC * NROW, HW), jnp.float32),
    mesh=_mesh,
    compiler_params=_sc_params,
    scratch_types=[
        pltpu.VMEM_SHARED((NROW, HW), jnp.float32),
        pltpu.VMEM((SCE,), jnp.int32),
        pltpu.VMEM((SCE,), jnp.int32),
        pltpu.VMEM((SCE,), jnp.int32),
        pltpu.VMEM((SCE,), jnp.float32),
        [pltpu.VMEM((K, HW), jnp.float32)] * 4,
        [pltpu.VMEM((K,), jnp.int32)] * 8,
        pltpu.VMEM((L, HW), jnp.float32),
        [pltpu.SemaphoreType.DMA] * 4,
    ],
)
def _sc_messages(xw2_h, src_h, dst_h, et_h, en_h, aggp_h, acc_sh, src_v,
                 dst_v, et_v, en_v, rowbufs, idxbufs, zb_v, sems):
    loA_v, hiA_v, loB_v, hiB_v = rowbufs
    glA_v, ghA_v, slA_v, shA_v, glB_v, ghB_v, slB_v, shB_v = idxbufs
    gsA, gsB, ssA, ssB = sems
    c = lax.axis_index("c")
    s = lax.axis_index("s")
    lo = c * NH
    zeros16 = jnp.zeros((L,), jnp.float32)
    iota = lax.iota(jnp.int32, L)
    for j in range(L):
        for t in range(HW // L):
            zb_v[j, pl.ds(t * L, L)] = zeros16

    def zchunk(t, carry):
        pltpu.sync_copy(zb_v, acc_sh.at[pl.ds(s * NPT + t * L, L)])
        return carry

    lax.fori_loop(0, NPT // L, zchunk, 0)
    plsc.subcore_barrier()

    def schunk(g, carry):
        ebase = s * EPT + g * SCE
        pltpu.sync_copy(src_h.at[pl.ds(ebase, SCE)], src_v)
        pltpu.sync_copy(dst_h.at[pl.ds(ebase, SCE)], dst_v)
        pltpu.sync_copy(et_h.at[pl.ds(ebase, SCE)], et_v)
        pltpu.sync_copy(en_h.at[pl.ds(ebase, SCE)], en_v)

        def make_idx(i, gl_v, gh_v, sl_v, sh_v):
            base = i * K
            for v in range(K // L):
                off = base + v * L
                sr = src_v[pl.ds(off, L)]
                e = et_v[pl.ds(off, L)]
                d = dst_v[pl.ds(off, L)]
                slot = g * SCE + off + iota
                loc = d - lo
                valid = (slot < EPT) & (loc >= 0) & (loc < NH)
                g2 = (e * N + sr) * 2
                gl_v[pl.ds(v * L, L)] = g2
                gh_v[pl.ds(v * L, L)] = g2 + 1
                srow = jnp.where(valid, loc, NH) * 2
                sl_v[pl.ds(v * L, L)] = srow
                sh_v[pl.ds(v * L, L)] = srow + 1

        def normalize(i, lo_v, hi_v):
            base = i * K
            for v in range(K // L):
                nvec = en_v[pl.ds(base + v * L, L)]
                for j in range(L):
                    bc = jnp.broadcast_to(nvec[j], (L,))
                    row = v * L + j
                    for t in range(HW // L):
                        sl = pl.ds(t * L, L)
                        lo_v[row, sl] = lo_v[row, sl] * bc
                        hi_v[row, sl] = hi_v[row, sl] * bc

        def pair(i2, carry2):
            iA = 2 * i2
            iB = 2 * i2 + 1
            make_idx(iA, glA_v, ghA_v, slA_v, shA_v)
            dA1 = pltpu.async_copy(xw2_h.at[glA_v], loA_v, gsA)
            dA2 = pltpu.async_copy(xw2_h.at[ghA_v], hiA_v, gsA)
            make_idx(iB, glB_v, ghB_v, slB_v, shB_v)
            dB1 = pltpu.async_copy(xw2_h.at[glB_v], loB_v, gsB)
            dB2 = pltpu.async_copy(xw2_h.at[ghB_v], hiB_v, gsB)
            dA1.wait()
            dA2.wait()
            normalize(iA, loA_v, hiA_v)
            sA1 = pltpu.async_copy(loA_v, acc_sh.at[slA_v], ssA, add=True)
            sA2 = pltpu.async_copy(hiA_v, acc_sh.at[shA_v], ssA, add=True)
            dB1.wait()
            dB2.wait()
            normalize(iB, loB_v, hiB_v)
            sB1 = pltpu.async_copy(loB_v, acc_sh.at[slB_v], ssB, add=True)
            sB2 = pltpu.async_copy(hiB_v, acc_sh.at[shB_v], ssB, add=True)
            sA1.wait()
            sA2.wait()
            sB1.wait()
            sB2.wait()
            return carry2

        lax.fori_loop(0, NCHI // 2, pair, 0)
        return carry

    lax.fori_loop(0, NSC, schunk, 0)
    plsc.subcore_barrier()

    def wchunk(t, carry):
        r0 = s * NPT + t * L
        pltpu.sync_copy(acc_sh.at[pl.ds(r0, L)], zb_v)
        pltpu.sync_copy(zb_v, aggp_h.at[pl.ds(c * NROW + r0, L)])
        return carry

    lax.fori_loop(0, NPT // L, wchunk, 0)


def _w_body(comp_ref, basis_ref, w_ref):
    w_ref[...] = jnp.dot(comp_ref[...], basis_ref[...],
                         preferred_element_type=jnp.float32)


def _relation_weights(comp, basis):
    nb = basis.shape[0]
    w2d = pl.pallas_call(
        _w_body,
        out_shape=jax.ShapeDtypeStruct((R, D * D), jnp.float32),
    )(comp, basis.reshape(nb, D * D))
    return w2d.reshape(R, D, D)


BN = 2000


def _xw_body(x_ref, w_ref, o_ref):
    o_ref[0] = jnp.dot(x_ref[...], w_ref[0],
                       preferred_element_type=jnp.float32)


def _all_relation_transform(x, w):
    return pl.pallas_call(
        _xw_body,
        grid=(R, N // BN),
        in_specs=[
            pl.BlockSpec((BN, D), lambda r, j: (j, 0)),
            pl.BlockSpec((1, D, D), lambda r, j: (r, 0, 0)),
        ],
        out_specs=pl.BlockSpec((1, BN, D), lambda r, j: (r, j, 0)),
        out_shape=jax.ShapeDtypeStruct((R, N, D), jnp.float32),
    )(x, w)


def _comb_body(agg_ref, x_ref, root_ref, bias_ref, o_ref):
    acc = agg_ref[...] + jnp.dot(x_ref[...], root_ref[...],
                                 preferred_element_type=jnp.float32)
    o_ref[...] = jnp.maximum(acc + bias_ref[...], 0.0)


def _combine(agg, x, root, bias):
    return pl.pallas_call(
        _comb_body,
        grid=(N // BN,),
        in_specs=[
            pl.BlockSpec((BN, D), lambda j: (j, 0)),
            pl.BlockSpec((BN, D), lambda j: (j, 0)),
            pl.BlockSpec((D, D), lambda j: (0, 0)),
            pl.BlockSpec((1, D), lambda j: (0, 0)),
        ],
        out_specs=pl.BlockSpec((BN, D), lambda j: (j, 0)),
        out_shape=jax.ShapeDtypeStruct((N, D), jnp.float32),
    )(agg, x, root, bias.reshape(1, D))


def _layer(x, srcp, dstp, etp, basis, comp, root, bias, en):
    w = _relation_weights(comp, basis)
    xw = _all_relation_transform(x, w)
    aggp = _sc_messages(xw.reshape(R * N * 2, HW), srcp, dstp, etp, en)
    agg = aggp.reshape(NC, NROW // 2, D)[:, :NH].reshape(N, D)
    return _combine(agg, x, root, bias)


def kernel(x, edge_index, edge_type, basis1, comp1, root1, bias1,
           basis2, comp2, root2, bias2):
    srcp = jnp.pad(edge_index[0], (0, EP2 - E))
    dstp = jnp.pad(edge_index[1], (0, EP2 - E))
    etp = jnp.pad(edge_type, (0, EP2 - E))
    norm = _sc_counts(dstp, etp)
    en = _sc_edge_norm(dstp, etp, norm)
    h = _layer(x, srcp, dstp, etp, basis1, comp1, root1, bias1, en)
    h = _layer(h, srcp, dstp, etp, basis2, comp2, root2, bias2, en)
    return h


# speedup vs baseline: 2.1936x; 1.2357x over previous
import functools

import jax
import jax.numpy as jnp
from jax import lax
from jax.experimental import pallas as pl
from jax.experimental.pallas import tpu as pltpu
from jax.experimental.pallas import tpu_sc as plsc

N = 10000
E = 160000
R = 8
D = 256

NC = 2
NS = 16
L = 16
HW = 128

NH = N // NC
KH = NH * R
KHP = 40960
KR = KHP // HW
KTRASH = 316
NRT = 32

NROW = 10240
NPT = NROW // NS

EPT = E // NS
SCE = 1024
NSC = 10
K = 64
NCHI = SCE // K
EP2 = (NS - 1) * EPT + NSC * SCE + 64

_mesh = plsc.VectorSubcoreMesh(
    core_axis_name="c", subcore_axis_name="s", num_cores=NC, num_subcores=NS)
_sc_params = pltpu.CompilerParams(needs_layout_passes=False)


@functools.partial(
    pl.kernel,
    out_type=jax.ShapeDtypeStruct((NC * KHP,), jnp.float32),
    mesh=_mesh,
    compiler_params=_sc_params,
    scratch_types=[
        pltpu.VMEM_SHARED((KR, HW), jnp.float32),
        pltpu.VMEM((SCE,), jnp.int32),
        pltpu.VMEM((SCE,), jnp.int32),
        pltpu.VMEM((K,), jnp.int32),
        pltpu.VMEM((K, HW), jnp.float32),
        pltpu.VMEM((NRT, HW), jnp.float32),
        pltpu.VMEM((NRT * HW,), jnp.float32),
    ],
)
def _sc_counts(dst_h, et_h, norm_h, acc_sh, dst_v, et_v, kidx_v, val_v,
               buf_v, nrm_v):
    c = lax.axis_index("c")
    s = lax.axis_index("s")
    lo = c * NH
    zeros16 = jnp.zeros((L,), jnp.float32)
    iota = lax.iota(jnp.int32, L)

    @pl.when(s < KR // NRT)
    def _zero():
        for j in range(NRT):
            for t in range(HW // L):
                buf_v[j, pl.ds(t * L, L)] = zeros16
        pltpu.sync_copy(buf_v, acc_sh.at[pl.ds(s * NRT, NRT)])

    plsc.subcore_barrier()

    def schunk(g, carry):
        ebase = s * EPT + g * SCE
        pltpu.sync_copy(dst_h.at[pl.ds(ebase, SCE)], dst_v)
        pltpu.sync_copy(et_h.at[pl.ds(ebase, SCE)], et_v)

        def chunk(i, carry2):
            base = i * K
            for v in range(K // L):
                off = base + v * L
                d = dst_v[pl.ds(off, L)]
                e = et_v[pl.ds(off, L)]
                slot = g * SCE + off + iota
                loc = d - lo
                valid = (slot < EPT) & (loc >= 0) & (loc < NH)
                lk = loc * R + e
                row = lax.shift_right_logical(lk, 7)
                lane = jnp.where(valid, lk & (HW - 1), 0)
                kidx_v[pl.ds(v * L, L)] = jnp.where(valid, row, KTRASH)
                for j in range(L):
                    bc = jnp.broadcast_to(lane[j], (L,))
                    for t in range(HW // L):
                        val_v[v * L + j, pl.ds(t * L, L)] = jnp.where(
                            iota + t * L == bc, 1.0, 0.0)
            pltpu.sync_copy(val_v, acc_sh.at[kidx_v], add=True)
            return carry2

        lax.fori_loop(0, NCHI, chunk, 0)
        return carry

    lax.fori_loop(0, NSC, schunk, 0)
    plsc.subcore_barrier()

    @pl.when(s < KR // NRT)
    def _readout():
        pltpu.sync_copy(acc_sh.at[pl.ds(s * NRT, NRT)], buf_v)
        for j in range(NRT):
            for t in range(HW // L):
                cnt = buf_v[j, pl.ds(t * L, L)]
                nrm_v[pl.ds(j * HW + t * L, L)] = (
                    1.0 / jnp.maximum(cnt, 1.0))
        pltpu.sync_copy(
            nrm_v, norm_h.at[pl.ds(c * KHP + s * (NRT * HW), NRT * HW)])


EPW = E // (NC * NS)


@functools.partial(
    pl.kernel,
    out_type=jax.ShapeDtypeStruct((EP2,), jnp.float32),
    mesh=_mesh,
    compiler_params=_sc_params,
    scratch_types=[
        pltpu.VMEM((NC * KHP,), jnp.float32),
        pltpu.VMEM((EPW,), jnp.int32),
        pltpu.VMEM((EPW,), jnp.int32),
        pltpu.VMEM((EPW,), jnp.float32),
    ],
)
def _sc_edge_norm(dst_h, et_h, norm_h, en_h, nrm_v, dst_v, et_v, out_v):
    c = lax.axis_index("c")
    s = lax.axis_index("s")
    w = c * NS + s
    pltpu.sync_copy(norm_h, nrm_v)
    pltpu.sync_copy(dst_h.at[pl.ds(w * EPW, EPW)], dst_v)
    pltpu.sync_copy(et_h.at[pl.ds(w * EPW, EPW)], et_v)

    def chunk(i, carry):
        base = jnp.minimum(i * K, EPW - K)
        for v in range(K // L):
            off = base + v * L
            d = dst_v[pl.ds(off, L)]
            e = et_v[pl.ds(off, L)]
            half = jnp.where(d >= NH, 1, 0)
            gk = half * KHP + (d - half * NH) * R + e
            out_v[pl.ds(off, L)] = plsc.load_gather(nrm_v, [gk])
        return carry

    lax.fori_loop(0, (EPW + K - 1) // K, chunk, 0)
    pltpu.sync_copy(out_v, en_h.at[pl.ds(w * EPW, EPW)])


@functools.partial(
    pl.kernel,
    out_type=jax.ShapeDtypeStruct((NC * NROW, HW), jnp.float32),
    mesh=_mesh,
    compiler_params=_sc_params,
    scratch_types=[
        pltpu.VMEM_SHARED((NROW, HW), jnp.float32),
        pltpu.VMEM((SCE,), jnp.int32),
        pltpu.VMEM((SCE,), jnp.int32),
        pltpu.VMEM((SCE,), jnp.int32),
        pltpu.VMEM((SCE,), jnp.float32),
        pltpu.VMEM((K, D), jnp.float32),
        pltpu.VMEM((2 * K, HW), jnp.float32),
        pltpu.VMEM((K,), jnp.int32),
        pltpu.VMEM((2 * K,), jnp.int32),
        pltpu.VMEM((L, HW), jnp.float32),
    ],
)
def _sc_messages(xw_h, src_h, dst_h, et_h, en_h, aggp_h, acc_sh, src_v,
                 dst_v, et_v, en_v, rows_v, sb_v, gidx_v, sidx2_v, zb_v):
    c = lax.axis_index("c")
    s = lax.axis_index("s")
    lo = c * NH
    zeros16 = jnp.zeros((L,), jnp.float32)
    iota = lax.iota(jnp.int32, L)
    for j in range(L):
        for t in range(HW // L):
            zb_v[j, pl.ds(t * L, L)] = zeros16

    def zchunk(t, carry):
        pltpu.sync_copy(zb_v, acc_sh.at[pl.ds(s * NPT + t * L, L)])
        return carry

    lax.fori_loop(0, NPT // L, zchunk, 0)
    plsc.subcore_barrier()

    def schunk(g, carry):
        ebase = s * EPT + g * SCE
        pltpu.sync_copy(src_h.at[pl.ds(ebase, SCE)], src_v)
        pltpu.sync_copy(dst_h.at[pl.ds(ebase, SCE)], dst_v)
        pltpu.sync_copy(et_h.at[pl.ds(ebase, SCE)], et_v)
        pltpu.sync_copy(en_h.at[pl.ds(ebase, SCE)], en_v)

        pat = iota // 2
        par = iota & 1

        def chunk(i, carry2):
            base = i * K
            for v in range(K // L):
                off = base + v * L
                sr = src_v[pl.ds(off, L)]
                e = et_v[pl.ds(off, L)]
                d = dst_v[pl.ds(off, L)]
                slot = g * SCE + off + iota
                loc = d - lo
                valid = (slot < EPT) & (loc >= 0) & (loc < NH)
                gidx_v[pl.ds(v * L, L)] = e * N + sr
                srow = jnp.where(valid, loc, NH)
                sidx2_v[pl.ds(2 * v * L, L)] = srow[pat] * 2 + par
                sidx2_v[pl.ds(2 * v * L + L, L)] = srow[pat + 8] * 2 + par
            pltpu.sync_copy(xw_h.at[gidx_v], rows_v)
            for v in range(K // L):
                nvec = en_v[pl.ds(base + v * L, L)]
                for j in range(L):
                    bc = jnp.broadcast_to(nvec[j], (L,))
                    row = v * L + j
                    for t in range(D // L):
                        dst_row = 2 * row + t // (HW // L)
                        dst_sl = pl.ds((t % (HW // L)) * L, L)
                        sb_v[dst_row, dst_sl] = (
                            rows_v[row, pl.ds(t * L, L)] * bc)
            pltpu.sync_copy(sb_v, acc_sh.at[sidx2_v], add=True)
            return carry2

        lax.fori_loop(0, NCHI, chunk, 0)
        return carry

    lax.fori_loop(0, NSC, schunk, 0)
    plsc.subcore_barrier()

    def wchunk(t, carry):
        r0 = s * NPT + t * L
        pltpu.sync_copy(acc_sh.at[pl.ds(r0, L)], zb_v)
        pltpu.sync_copy(zb_v, aggp_h.at[pl.ds(c * NROW + r0, L)])
        return carry

    lax.fori_loop(0, NPT // L, wchunk, 0)


def _w_body(comp_ref, basis_ref, w_ref):
    w_ref[...] = jnp.dot(comp_ref[...], basis_ref[...],
                         preferred_element_type=jnp.float32)


def _relation_weights(comp, basis):
    nb = basis.shape[0]
    w2d = pl.pallas_call(
        _w_body,
        out_shape=jax.ShapeDtypeStruct((R, D * D), jnp.float32),
    )(comp, basis.reshape(nb, D * D))
    return w2d.reshape(R, D, D)


BN = 2000


def _xw_body(x_ref, w_ref, o_ref):
    o_ref[0] = jnp.dot(x_ref[...], w_ref[0],
                       preferred_element_type=jnp.float32)


def _all_relation_transform(x, w):
    return pl.pallas_call(
        _xw_body,
        grid=(R, N // BN),
        in_specs=[
            pl.BlockSpec((BN, D), lambda r, j: (j, 0)),
            pl.BlockSpec((1, D, D), lambda r, j: (r, 0, 0)),
        ],
        out_specs=pl.BlockSpec((1, BN, D), lambda r, j: (r, j, 0)),
        out_shape=jax.ShapeDtypeStruct((R, N, D), jnp.float32),
    )(x, w)


def _comb_body(agg_ref, x_ref, root_ref, bias_ref, o_ref):
    acc = agg_ref[...] + jnp.dot(x_ref[...], root_ref[...],
                                 preferred_element_type=jnp.float32)
    o_ref[...] = jnp.maximum(acc + bias_ref[...], 0.0)


def _combine(agg, x, root, bias):
    return pl.pallas_call(
        _comb_body,
        grid=(N // BN,),
        in_specs=[
            pl.BlockSpec((BN, D), lambda j: (j, 0)),
            pl.BlockSpec((BN, D), lambda j: (j, 0)),
            pl.BlockSpec((D, D), lambda j: (0, 0)),
            pl.BlockSpec((1, D), lambda j: (0, 0)),
        ],
        out_specs=pl.BlockSpec((BN, D), lambda j: (j, 0)),
        out_shape=jax.ShapeDtypeStruct((N, D), jnp.float32),
    )(agg, x, root, bias.reshape(1, D))


def _layer(x, srcp, dstp, etp, basis, comp, root, bias, en):
    w = _relation_weights(comp, basis)
    xw = _all_relation_transform(x, w)
    aggp = _sc_messages(xw.reshape(R * N, D), srcp, dstp, etp, en)
    agg = aggp.reshape(NC, NROW // 2, D)[:, :NH].reshape(N, D)
    return _combine(agg, x, root, bias)


def kernel(x, edge_index, edge_type, basis1, comp1, root1, bias1,
           basis2, comp2, root2, bias2):
    srcp = jnp.pad(edge_index[0], (0, EP2 - E))
    dstp = jnp.pad(edge_index[1], (0, EP2 - E))
    etp = jnp.pad(edge_type, (0, EP2 - E))
    norm = _sc_counts(dstp, etp)
    en = _sc_edge_norm(dstp, etp, norm)
    h = _layer(x, srcp, dstp, etp, basis1, comp1, root1, bias1, en)
    h = _layer(h, srcp, dstp, etp, basis2, comp2, root2, bias2, en)
    return h
